# baseline (speedup 1.0000x reference)
import jax
import jax.numpy as jnp
from jax import lax
from jax.experimental import pallas as pl
from jax.experimental.pallas import tpu as pltpu
from jax.experimental.pallas import tpu_sc as plsc

_NU = 25000
_NI = 25000
_N = _NU + _NI
_D = 64
_DH = _D // 2
_E = 400000

_TILES = 16
_CHUNK = 128
_STEPS = 200
_ROWS_PER_CORE = _TILES * _STEPS
_EPAD = _ROWS_PER_CORE * _CHUNK
_TRASH = _NU

_ACC_ROWS = 25088
_TROWS = _ACC_ROWS // _TILES
_WBCH = 112
_WBN = _TROWS // _WBCH
_TAILR = _NU - (15 * _TROWS + 13 * _WBCH)

_SEC = 5
_SSTEPS = _STEPS // _SEC

_mesh = plsc.VectorSubcoreMesh(core_axis_name="c", subcore_axis_name="s")
_sc_params = pltpu.CompilerParams(use_tc_tiling_on_sc=False,
                                  needs_layout_passes=False)


def _rsqrt16(x):
    xi = plsc.bitcast(x, jnp.int32)
    y = plsc.bitcast(1597463007 - (xi >> 1), jnp.float32)
    y = y * (1.5 - 0.5 * x * y * y)
    y = y * (1.5 - 0.5 * x * y * y)
    return jnp.where(x > 0.0, y, 0.0)


def _hist_body(sidx_hbm, dinv_hbm, sidx_v, ones_v, zbuf_v, deg_sh):
    c = lax.axis_index("c")
    s = lax.axis_index("s")
    z16 = jnp.zeros((16,), jnp.float32)
    one16 = jnp.ones((16,), jnp.float32)

    @pl.loop(0, _WBCH)
    def _(i):
        zbuf_v[i, :] = z16

    @pl.loop(0, _CHUNK)
    def _(i):
        ones_v[i, :] = one16

    @pl.loop(0, _WBN)
    def _(k):
        pltpu.sync_copy(zbuf_v, deg_sh.at[pl.ds(s * _TROWS + k * _WBCH, _WBCH)])

    pltpu.sync_copy(sidx_hbm.at[c, pl.ds(s * _STEPS, _STEPS)], sidx_v)
    plsc.subcore_barrier()

    @pl.loop(0, _STEPS)
    def _(j):
        pltpu.sync_copy(ones_v, deg_sh.at[sidx_v.at[j]], add=True)

    plsc.subcore_barrier()

    @pl.loop(0, _WBN)
    def _(k):
        r = s * _TROWS + k * _WBCH
        pltpu.sync_copy(deg_sh.at[pl.ds(r, _WBCH)], zbuf_v)

        @pl.loop(0, _WBCH)
        def _(i):
            zbuf_v[i, :] = _rsqrt16(zbuf_v[i, :])

        pltpu.sync_copy(zbuf_v, dinv_hbm.at[c, pl.ds(r, _WBCH)])


def _spmm_body(sidx_hbm, w_hbm, dinv_hbm, acc_hbm,
               gidx_v, sidx_v, rows0, rows1, wbuf, dvbuf,
               tab_sh, acc_sh, sem0, sem1, ssem0, ssem1):
    c = lax.axis_index("c")
    s = lax.axis_index("s")
    z16 = jnp.zeros((16,), jnp.float32)

    def _gather(j, buf, sem):
        pltpu.async_copy(tab_sh.at[gidx_v.at[j]], buf, sem)

    def _wait(buf, sem):
        pltpu.make_async_copy(tab_sh.at[gidx_v.at[0]], buf, sem).wait()

    def _scatter_async(j, buf, sem):
        pltpu.async_copy(buf, acc_sh.at[sidx_v.at[j]], sem, add=True)

    def _swait(buf, sem):
        pltpu.make_async_copy(buf, acc_sh.at[sidx_v.at[0]], sem).wait()

    def _scatter(j, buf):
        pltpu.sync_copy(buf, acc_sh.at[sidx_v.at[j]], add=True)

    for h in (0, 1):
        def _stage(nrows, r0):
            pltpu.sync_copy(w_hbm.at[pl.ds((1 - c) * _NU + r0, nrows)],
                            wbuf.at[pl.ds(0, nrows)])
            pltpu.sync_copy(dinv_hbm.at[1 - c, pl.ds(r0, nrows)],
                            dvbuf.at[pl.ds(0, nrows)])

            @pl.loop(0, nrows)
            def _(i):
                dv = dvbuf[i, :] * 0.75
                for q in range(_DH // 16):
                    rows0[i, pl.ds(q * 16, 16)] = (
                        wbuf[i, pl.ds(h * _DH + q * 16, 16)] * dv)

            pltpu.sync_copy(rows0.at[pl.ds(0, nrows)], tab_sh.at[pl.ds(r0, nrows)])

        @pl.loop(0, _WBN)
        def _(k):
            r0 = s * _TROWS + k * _WBCH
            last = jnp.logical_and(s == _TILES - 1, k == _WBN - 1)

            @pl.when(jnp.logical_not(last))
            def _():
                _stage(_WBCH, r0)

            @pl.when(last)
            def _():
                _stage(_TAILR, r0)

        @pl.loop(0, _WBCH)
        def _(i):
            for q in range(_DH // 16):
                rows0[i, pl.ds(q * 16, 16)] = z16

        @pl.loop(0, _WBN)
        def _(k):
            pltpu.sync_copy(rows0.at[pl.ds(0, _WBCH)],
                            acc_sh.at[pl.ds(s * _TROWS + k * _WBCH, _WBCH)])

        plsc.subcore_barrier()

        @pl.loop(0, _SEC)
        def _(sec):
            base = pl.ds(s * _STEPS + sec * _SSTEPS, _SSTEPS)
            pltpu.sync_copy(sidx_hbm.at[1 - c, base], gidx_v)
            pltpu.sync_copy(sidx_hbm.at[c, base], sidx_v)

            _gather(0, rows0, sem0)
            _gather(1, rows1, sem1)

            @pl.loop(0, _SSTEPS // 2 - 1)
            def _(t):
                j = 2 * t
                _wait(rows0, sem0)
                _scatter_async(j, rows0, ssem0)
                _wait(rows1, sem1)
                _scatter_async(j + 1, rows1, ssem1)
                _swait(rows0, ssem0)
                _gather(j + 2, rows0, sem0)
                _swait(rows1, ssem1)
                _gather(j + 3, rows1, sem1)

            _wait(rows0, sem0)
            _scatter(_SSTEPS - 2, rows0)
            _wait(rows1, sem1)
            _scatter(_SSTEPS - 1, rows1)

        plsc.subcore_barrier()

        @pl.loop(0, _WBN)
        def _(k):
            r = s * _TROWS + k * _WBCH
            pltpu.sync_copy(acc_sh.at[pl.ds(r, _WBCH)], rows0.at[pl.ds(0, _WBCH)])
            pltpu.sync_copy(rows0.at[pl.ds(0, _WBCH)],
                            acc_hbm.at[c, h, pl.ds(r, _WBCH)])


_hist = pl.kernel(
    _hist_body,
    out_type=jax.ShapeDtypeStruct((2, _ACC_ROWS, 16), jnp.float32),
    mesh=_mesh,
    scratch_types=[
        pltpu.VMEM((_STEPS, _CHUNK), jnp.int32),
        pltpu.VMEM((_CHUNK, 16), jnp.float32),
        pltpu.VMEM((_WBCH, 16), jnp.float32),
        pltpu.VMEM_SHARED((_ACC_ROWS, 16), jnp.float32),
    ],
    compiler_params=_sc_params,
)

_spmm = pl.kernel(
    _spmm_body,
    out_type=jax.ShapeDtypeStruct((2, 2, _ACC_ROWS, _DH), jnp.float32),
    mesh=_mesh,
    scratch_types=[
        pltpu.VMEM((_SSTEPS, _CHUNK), jnp.int32),
        pltpu.VMEM((_SSTEPS, _CHUNK), jnp.int32),
        pltpu.VMEM((_CHUNK, _DH), jnp.float32),
        pltpu.VMEM((_CHUNK, _DH), jnp.float32),
        pltpu.VMEM((_WBCH, _D), jnp.float32),
        pltpu.VMEM((_WBCH, 16), jnp.float32),
        pltpu.VMEM_SHARED((_ACC_ROWS, _DH), jnp.float32),
        pltpu.VMEM_SHARED((_ACC_ROWS, _DH), jnp.float32),
        pltpu.SemaphoreType.DMA,
        pltpu.SemaphoreType.DMA,
        pltpu.SemaphoreType.DMA,
        pltpu.SemaphoreType.DMA,
    ],
    compiler_params=_sc_params,
)

_BLK = 5000
_NBLK = _NU // _BLK


def _blend_body(wu_ref, wi_ref, du_ref, di_ref,
                au0_ref, au1_ref, ai0_ref, ai1_ref, ou_ref, oi_ref):
    dinvu = du_ref[0, :, :1]
    dinvi = di_ref[0, :, :1]
    au = jnp.concatenate([au0_ref[0, 0, :, :], au1_ref[0, 0, :, :]], axis=1)
    ai = jnp.concatenate([ai0_ref[0, 0, :, :], ai1_ref[0, 0, :, :]], axis=1)
    ou_ref[:, :] = 0.25 * wu_ref[:, :] + dinvu * au
    oi_ref[:, :] = 0.25 * wi_ref[:, :] + dinvi * ai


_blend = pl.pallas_call(
    _blend_body,
    grid=(_NBLK,),
    in_specs=[
        pl.BlockSpec((_BLK, _D), lambda i: (i, 0)),
        pl.BlockSpec((_BLK, _D), lambda i: (i + _NBLK, 0)),
        pl.BlockSpec((1, _BLK, 16), lambda i: (0, i, 0)),
        pl.BlockSpec((1, _BLK, 16), lambda i: (1, i, 0)),
        pl.BlockSpec((1, 1, _BLK, _DH), lambda i: (0, 0, i, 0)),
        pl.BlockSpec((1, 1, _BLK, _DH), lambda i: (0, 1, i, 0)),
        pl.BlockSpec((1, 1, _BLK, _DH), lambda i: (1, 0, i, 0)),
        pl.BlockSpec((1, 1, _BLK, _DH), lambda i: (1, 1, i, 0)),
    ],
    out_specs=[
        pl.BlockSpec((_BLK, _D), lambda i: (i, 0)),
        pl.BlockSpec((_BLK, _D), lambda i: (i, 0)),
    ],
    out_shape=[
        jax.ShapeDtypeStruct((_NU, _D), jnp.float32),
        jax.ShapeDtypeStruct((_NI, _D), jnp.float32),
    ],
)


def kernel(embed_weight, user_idxs, user_id_idx, item_id_idx):
    del user_idxs
    w = embed_weight
    npad = _EPAD - _E
    pad_s = jnp.full((npad,), _TRASH, jnp.int32)
    sidx = jnp.stack([
        jnp.concatenate([user_id_idx, pad_s]),
        jnp.concatenate([item_id_idx, pad_s]),
    ]).reshape(2, _ROWS_PER_CORE, _CHUNK)

    dinv = _hist(sidx)
    acc = _spmm(sidx, w, dinv)
    user_embed, item_embed = _blend(w, w, dinv, dinv, acc, acc, acc, acc)
    return (user_embed, item_embed)


# speedup vs baseline: 1.0635x; 1.0635x over previous
import jax
import jax.numpy as jnp
from jax import lax
from jax.experimental import pallas as pl
from jax.experimental.pallas import tpu as pltpu
from jax.experimental.pallas import tpu_sc as plsc

_NU = 25000
_NI = 25000
_N = _NU + _NI
_D = 64
_DH = _D // 2
_E = 400000

_TILES = 16
_CHUNK = 128
_STEPS = 200
_ROWS_PER_CORE = _TILES * _STEPS
_EPAD = _ROWS_PER_CORE * _CHUNK
_TRASH = _NU

_ACC_ROWS = 25088
_TROWS = _ACC_ROWS // _TILES
_WBCH = 112
_WBN = _TROWS // _WBCH
_TAILR = _NU - (15 * _TROWS + 13 * _WBCH)

_SEC = 5
_SSTEPS = _STEPS // _SEC

_mesh = plsc.VectorSubcoreMesh(core_axis_name="c", subcore_axis_name="s")
_sc_params = pltpu.CompilerParams(use_tc_tiling_on_sc=False,
                                  needs_layout_passes=False)


def _rsqrt16(x):
    xi = plsc.bitcast(x, jnp.int32)
    y = plsc.bitcast(1597463007 - (xi >> 1), jnp.float32)
    y = y * (1.5 - 0.5 * x * y * y)
    y = y * (1.5 - 0.5 * x * y * y)
    return jnp.where(x > 0.0, y, 0.0)


def _hist_body(sidx_hbm, dinv_hbm, sidx_v, ones_v, zbuf_v, deg_sh):
    c = lax.axis_index("c")
    s = lax.axis_index("s")
    z16 = jnp.zeros((16,), jnp.float32)
    one16 = jnp.ones((16,), jnp.float32)

    @pl.loop(0, _WBCH)
    def _(i):
        zbuf_v[i, :] = z16

    @pl.loop(0, _CHUNK)
    def _(i):
        ones_v[i, :] = one16

    @pl.loop(0, _WBN)
    def _(k):
        pltpu.sync_copy(zbuf_v, deg_sh.at[pl.ds(s * _TROWS + k * _WBCH, _WBCH)])

    pltpu.sync_copy(sidx_hbm.at[c, pl.ds(s * _STEPS, _STEPS)], sidx_v)
    plsc.subcore_barrier()

    @pl.loop(0, _STEPS)
    def _(j):
        pltpu.sync_copy(ones_v, deg_sh.at[sidx_v.at[j]], add=True)

    plsc.subcore_barrier()

    @pl.loop(0, _WBN)
    def _(k):
        r = s * _TROWS + k * _WBCH
        pltpu.sync_copy(deg_sh.at[pl.ds(r, _WBCH)], zbuf_v)

        @pl.loop(0, _WBCH)
        def _(i):
            zbuf_v[i, :] = _rsqrt16(zbuf_v[i, :])

        pltpu.sync_copy(zbuf_v, dinv_hbm.at[c, pl.ds(r, _WBCH)])


def _spmm_body(sidx_hbm, w_hbm, dinv_hbm, acc_hbm,
               gidx_v, sidx_v, rows0, rows1, wbuf, dvbuf,
               tab_sh, acc_sh, sem0, sem1, ssem0, ssem1):
    c = lax.axis_index("c")
    s = lax.axis_index("s")
    z16 = jnp.zeros((16,), jnp.float32)

    def _gather(j, buf, sem):
        pltpu.async_copy(tab_sh.at[gidx_v.at[j]], buf, sem)

    def _wait(buf, sem):
        pltpu.make_async_copy(tab_sh.at[gidx_v.at[0]], buf, sem).wait()

    def _scatter_async(j, buf, sem):
        pltpu.async_copy(buf, acc_sh.at[sidx_v.at[j]], sem, add=True)

    def _swait(buf, sem):
        pltpu.make_async_copy(buf, acc_sh.at[sidx_v.at[0]], sem).wait()

    def _scatter(j, buf):
        pltpu.sync_copy(buf, acc_sh.at[sidx_v.at[j]], add=True)

    for h in (0, 1):
        def _stage(nrows, r0):
            pltpu.sync_copy(w_hbm.at[pl.ds((1 - c) * _NU + r0, nrows)],
                            wbuf.at[pl.ds(0, nrows)])
            pltpu.sync_copy(dinv_hbm.at[1 - c, pl.ds(r0, nrows)],
                            dvbuf.at[pl.ds(0, nrows)])

            @pl.loop(0, nrows)
            def _(i):
                dv = dvbuf[i, :] * 0.75
                for q in range(_DH // 16):
                    rows0[i, pl.ds(q * 16, 16)] = (
                        wbuf[i, pl.ds(h * _DH + q * 16, 16)] * dv)

            pltpu.sync_copy(rows0.at[pl.ds(0, nrows)], tab_sh.at[pl.ds(r0, nrows)])

        @pl.loop(0, _WBN)
        def _(k):
            r0 = s * _TROWS + k * _WBCH
            last = jnp.logical_and(s == _TILES - 1, k == _WBN - 1)

            @pl.when(jnp.logical_not(last))
            def _():
                _stage(_WBCH, r0)

            @pl.when(last)
            def _():
                _stage(_TAILR, r0)

        @pl.loop(0, _WBCH)
        def _(i):
            for q in range(_DH // 16):
                rows0[i, pl.ds(q * 16, 16)] = z16

        @pl.loop(0, _WBN)
        def _(k):
            pltpu.sync_copy(rows0.at[pl.ds(0, _WBCH)],
                            acc_sh.at[pl.ds(s * _TROWS + k * _WBCH, _WBCH)])

        plsc.subcore_barrier()

        @pl.loop(0, _SEC)
        def _(sec):
            base = pl.ds(s * _STEPS + sec * _SSTEPS, _SSTEPS)
            pltpu.sync_copy(sidx_hbm.at[1 - c, base], gidx_v)
            pltpu.sync_copy(sidx_hbm.at[c, base], sidx_v)

            _gather(0, rows0, sem0)

            @pl.loop(0, _SSTEPS // 2 - 1)
            def _(t):
                j = 2 * t
                _gather(j + 1, rows1, sem1)
                _wait(rows0, sem0)
                _scatter(j, rows0)
                _gather(j + 2, rows0, sem0)
                _wait(rows1, sem1)
                _scatter(j + 1, rows1)

            _gather(_SSTEPS - 1, rows1, sem1)
            _wait(rows0, sem0)
            _scatter(_SSTEPS - 2, rows0)
            _wait(rows1, sem1)
            _scatter(_SSTEPS - 1, rows1)

        plsc.subcore_barrier()

        @pl.loop(0, _WBN)
        def _(k):
            r = s * _TROWS + k * _WBCH
            pltpu.sync_copy(acc_sh.at[pl.ds(r, _WBCH)], rows0.at[pl.ds(0, _WBCH)])
            pltpu.sync_copy(rows0.at[pl.ds(0, _WBCH)],
                            acc_hbm.at[c, h, pl.ds(r, _WBCH)])


_hist = pl.kernel(
    _hist_body,
    out_type=jax.ShapeDtypeStruct((2, _ACC_ROWS, 16), jnp.float32),
    mesh=_mesh,
    scratch_types=[
        pltpu.VMEM((_STEPS, _CHUNK), jnp.int32),
        pltpu.VMEM((_CHUNK, 16), jnp.float32),
        pltpu.VMEM((_WBCH, 16), jnp.float32),
        pltpu.VMEM_SHARED((_ACC_ROWS, 16), jnp.float32),
    ],
    compiler_params=_sc_params,
)

_spmm = pl.kernel(
    _spmm_body,
    out_type=jax.ShapeDtypeStruct((2, 2, _ACC_ROWS, _DH), jnp.float32),
    mesh=_mesh,
    scratch_types=[
        pltpu.VMEM((_SSTEPS, _CHUNK), jnp.int32),
        pltpu.VMEM((_SSTEPS, _CHUNK), jnp.int32),
        pltpu.VMEM((_CHUNK, _DH), jnp.float32),
        pltpu.VMEM((_CHUNK, _DH), jnp.float32),
        pltpu.VMEM((_WBCH, _D), jnp.float32),
        pltpu.VMEM((_WBCH, 16), jnp.float32),
        pltpu.VMEM_SHARED((_ACC_ROWS, _DH), jnp.float32),
        pltpu.VMEM_SHARED((_ACC_ROWS, _DH), jnp.float32),
        pltpu.SemaphoreType.DMA,
        pltpu.SemaphoreType.DMA,
        pltpu.SemaphoreType.DMA,
        pltpu.SemaphoreType.DMA,
    ],
    compiler_params=_sc_params,
)

_BLK = 5000
_NBLK = _NU // _BLK


def _blend_body(wu_ref, wi_ref, du_ref, di_ref,
                au0_ref, au1_ref, ai0_ref, ai1_ref, ou_ref, oi_ref):
    dinvu = du_ref[0, :, :1]
    dinvi = di_ref[0, :, :1]
    au = jnp.concatenate([au0_ref[0, 0, :, :], au1_ref[0, 0, :, :]], axis=1)
    ai = jnp.concatenate([ai0_ref[0, 0, :, :], ai1_ref[0, 0, :, :]], axis=1)
    ou_ref[:, :] = 0.25 * wu_ref[:, :] + dinvu * au
    oi_ref[:, :] = 0.25 * wi_ref[:, :] + dinvi * ai


_blend = pl.pallas_call(
    _blend_body,
    grid=(_NBLK,),
    in_specs=[
        pl.BlockSpec((_BLK, _D), lambda i: (i, 0)),
        pl.BlockSpec((_BLK, _D), lambda i: (i + _NBLK, 0)),
        pl.BlockSpec((1, _BLK, 16), lambda i: (0, i, 0)),
        pl.BlockSpec((1, _BLK, 16), lambda i: (1, i, 0)),
        pl.BlockSpec((1, 1, _BLK, _DH), lambda i: (0, 0, i, 0)),
        pl.BlockSpec((1, 1, _BLK, _DH), lambda i: (0, 1, i, 0)),
        pl.BlockSpec((1, 1, _BLK, _DH), lambda i: (1, 0, i, 0)),
        pl.BlockSpec((1, 1, _BLK, _DH), lambda i: (1, 1, i, 0)),
    ],
    out_specs=[
        pl.BlockSpec((_BLK, _D), lambda i: (i, 0)),
        pl.BlockSpec((_BLK, _D), lambda i: (i, 0)),
    ],
    out_shape=[
        jax.ShapeDtypeStruct((_NU, _D), jnp.float32),
        jax.ShapeDtypeStruct((_NI, _D), jnp.float32),
    ],
)


def kernel(embed_weight, user_idxs, user_id_idx, item_id_idx):
    del user_idxs
    w = embed_weight
    npad = _EPAD - _E
    pad_s = jnp.full((npad,), _TRASH, jnp.int32)
    sidx = jnp.stack([
        jnp.concatenate([user_id_idx, pad_s]),
        jnp.concatenate([item_id_idx, pad_s]),
    ]).reshape(2, _ROWS_PER_CORE, _CHUNK)

    dinv = _hist(sidx)
    acc = _spmm(sidx, w, dinv)
    user_embed, item_embed = _blend(w, w, dinv, dinv, acc, acc, acc, acc)
    return (user_embed, item_embed)


# speedup vs baseline: 1.0715x; 1.0075x over previous
import jax
import jax.numpy as jnp
from jax import lax
from jax.experimental import pallas as pl
from jax.experimental.pallas import tpu as pltpu
from jax.experimental.pallas import tpu_sc as plsc

_NU = 25000
_NI = 25000
_N = _NU + _NI
_D = 64
_DH = _D // 2
_E = 400000

_TILES = 16
_CHUNK = 128
_STEPS = 200
_ROWS_PER_CORE = _TILES * _STEPS
_EPAD = _ROWS_PER_CORE * _CHUNK
_TRASH = _NU

_ACC_ROWS = 25088
_TROWS = _ACC_ROWS // _TILES
_WBCH = 112
_WBN = _TROWS // _WBCH
_TAILR = _NU - (15 * _TROWS + 13 * _WBCH)

_SEC = 5
_SSTEPS = _STEPS // _SEC

_mesh = plsc.VectorSubcoreMesh(core_axis_name="c", subcore_axis_name="s")
_sc_params = pltpu.CompilerParams(use_tc_tiling_on_sc=False,
                                  needs_layout_passes=False)


def _rsqrt16(x):
    xi = plsc.bitcast(x, jnp.int32)
    y = plsc.bitcast(1597463007 - (xi >> 1), jnp.float32)
    y = y * (1.5 - 0.5 * x * y * y)
    y = y * (1.5 - 0.5 * x * y * y)
    return jnp.where(x > 0.0, y, 0.0)


def _hist_body(sidx_hbm, dinv_hbm, sidx_v, ones_v, zbuf_v, deg_sh, hs0, hs1):
    c = lax.axis_index("c")
    s = lax.axis_index("s")
    z16 = jnp.zeros((16,), jnp.float32)
    one16 = jnp.ones((16,), jnp.float32)

    @pl.loop(0, _WBCH)
    def _(i):
        zbuf_v[i, :] = z16

    @pl.loop(0, _CHUNK)
    def _(i):
        ones_v[i, :] = one16

    @pl.loop(0, _WBN)
    def _(k):
        pltpu.sync_copy(zbuf_v, deg_sh.at[pl.ds(s * _TROWS + k * _WBCH, _WBCH)])

    pltpu.sync_copy(sidx_hbm.at[c, pl.ds(s * _STEPS, _STEPS)], sidx_v)
    plsc.subcore_barrier()

    def _ones(j, sem):
        pltpu.async_copy(ones_v, deg_sh.at[sidx_v.at[j]], sem, add=True)

    def _owait(sem):
        pltpu.make_async_copy(ones_v, deg_sh.at[sidx_v.at[0]], sem).wait()

    _ones(0, hs0)

    @pl.loop(0, _STEPS // 2 - 1)
    def _(t):
        j = 2 * t
        _ones(j + 1, hs1)
        _owait(hs0)
        _ones(j + 2, hs0)
        _owait(hs1)

    _ones(_STEPS - 1, hs1)
    _owait(hs0)
    _owait(hs1)

    plsc.subcore_barrier()

    @pl.loop(0, _WBN)
    def _(k):
        r = s * _TROWS + k * _WBCH
        pltpu.sync_copy(deg_sh.at[pl.ds(r, _WBCH)], zbuf_v)

        @pl.loop(0, _WBCH)
        def _(i):
            zbuf_v[i, :] = _rsqrt16(zbuf_v[i, :])

        pltpu.sync_copy(zbuf_v, dinv_hbm.at[c, pl.ds(r, _WBCH)])


def _spmm_body(sidx_hbm, w_hbm, dinv_hbm, acc_hbm,
               gidx_v, sidx_v, rows0, rows1, wbuf, dvbuf,
               tab_sh, acc_sh, sem0, sem1, ssem0, ssem1):
    c = lax.axis_index("c")
    s = lax.axis_index("s")
    z16 = jnp.zeros((16,), jnp.float32)

    def _gather(j, buf, sem):
        pltpu.async_copy(tab_sh.at[gidx_v.at[j]], buf, sem)

    def _wait(buf, sem):
        pltpu.make_async_copy(tab_sh.at[gidx_v.at[0]], buf, sem).wait()

    def _scatter_async(j, buf, sem):
        pltpu.async_copy(buf, acc_sh.at[sidx_v.at[j]], sem, add=True)

    def _swait(buf, sem):
        pltpu.make_async_copy(buf, acc_sh.at[sidx_v.at[0]], sem).wait()

    def _scatter(j, buf):
        pltpu.sync_copy(buf, acc_sh.at[sidx_v.at[j]], add=True)

    for h in (0, 1):
        def _stage(nrows, r0):
            pltpu.sync_copy(w_hbm.at[pl.ds((1 - c) * _NU + r0, nrows)],
                            wbuf.at[pl.ds(0, nrows)])
            pltpu.sync_copy(dinv_hbm.at[1 - c, pl.ds(r0, nrows)],
                            dvbuf.at[pl.ds(0, nrows)])

            @pl.loop(0, nrows)
            def _(i):
                dv = dvbuf[i, :] * 0.75
                for q in range(_DH // 16):
                    rows0[i, pl.ds(q * 16, 16)] = (
                        wbuf[i, pl.ds(h * _DH + q * 16, 16)] * dv)

            pltpu.sync_copy(rows0.at[pl.ds(0, nrows)], tab_sh.at[pl.ds(r0, nrows)])

        @pl.loop(0, _WBN)
        def _(k):
            r0 = s * _TROWS + k * _WBCH
            last = jnp.logical_and(s == _TILES - 1, k == _WBN - 1)

            @pl.when(jnp.logical_not(last))
            def _():
                _stage(_WBCH, r0)

            @pl.when(last)
            def _():
                _stage(_TAILR, r0)

        @pl.loop(0, _WBCH)
        def _(i):
            for q in range(_DH // 16):
                rows0[i, pl.ds(q * 16, 16)] = z16

        @pl.loop(0, _WBN)
        def _(k):
            pltpu.sync_copy(rows0.at[pl.ds(0, _WBCH)],
                            acc_sh.at[pl.ds(s * _TROWS + k * _WBCH, _WBCH)])

        plsc.subcore_barrier()

        @pl.loop(0, _SEC)
        def _(sec):
            base = pl.ds(s * _STEPS + sec * _SSTEPS, _SSTEPS)
            pltpu.sync_copy(sidx_hbm.at[1 - c, base], gidx_v)
            pltpu.sync_copy(sidx_hbm.at[c, base], sidx_v)

            _gather(0, rows0, sem0)

            @pl.loop(0, _SSTEPS // 2 - 1)
            def _(t):
                j = 2 * t
                _gather(j + 1, rows1, sem1)
                _wait(rows0, sem0)
                _scatter(j, rows0)
                _gather(j + 2, rows0, sem0)
                _wait(rows1, sem1)
                _scatter(j + 1, rows1)

            _gather(_SSTEPS - 1, rows1, sem1)
            _wait(rows0, sem0)
            _scatter(_SSTEPS - 2, rows0)
            _wait(rows1, sem1)
            _scatter(_SSTEPS - 1, rows1)

        plsc.subcore_barrier()

        @pl.loop(0, _WBN)
        def _(k):
            r = s * _TROWS + k * _WBCH
            pltpu.sync_copy(acc_sh.at[pl.ds(r, _WBCH)], rows0.at[pl.ds(0, _WBCH)])
            pltpu.sync_copy(rows0.at[pl.ds(0, _WBCH)],
                            acc_hbm.at[c, h, pl.ds(r, _WBCH)])


_hist = pl.kernel(
    _hist_body,
    out_type=jax.ShapeDtypeStruct((2, _ACC_ROWS, 16), jnp.float32),
    mesh=_mesh,
    scratch_types=[
        pltpu.VMEM((_STEPS, _CHUNK), jnp.int32),
        pltpu.VMEM((_CHUNK, 16), jnp.float32),
        pltpu.VMEM((_WBCH, 16), jnp.float32),
        pltpu.VMEM_SHARED((_ACC_ROWS, 16), jnp.float32),
        pltpu.SemaphoreType.DMA,
        pltpu.SemaphoreType.DMA,
    ],
    compiler_params=_sc_params,
)

_spmm = pl.kernel(
    _spmm_body,
    out_type=jax.ShapeDtypeStruct((2, 2, _ACC_ROWS, _DH), jnp.float32),
    mesh=_mesh,
    scratch_types=[
        pltpu.VMEM((_SSTEPS, _CHUNK), jnp.int32),
        pltpu.VMEM((_SSTEPS, _CHUNK), jnp.int32),
        pltpu.VMEM((_CHUNK, _DH), jnp.float32),
        pltpu.VMEM((_CHUNK, _DH), jnp.float32),
        pltpu.VMEM((_WBCH, _D), jnp.float32),
        pltpu.VMEM((_WBCH, 16), jnp.float32),
        pltpu.VMEM_SHARED((_ACC_ROWS, _DH), jnp.float32),
        pltpu.VMEM_SHARED((_ACC_ROWS, _DH), jnp.float32),
        pltpu.SemaphoreType.DMA,
        pltpu.SemaphoreType.DMA,
        pltpu.SemaphoreType.DMA,
        pltpu.SemaphoreType.DMA,
    ],
    compiler_params=_sc_params,
)

_BLK = 5000
_NBLK = _NU // _BLK


def _blend_body(wu_ref, wi_ref, du_ref, di_ref,
                au0_ref, au1_ref, ai0_ref, ai1_ref, ou_ref, oi_ref):
    dinvu = du_ref[0, :, :1]
    dinvi = di_ref[0, :, :1]
    au = jnp.concatenate([au0_ref[0, 0, :, :], au1_ref[0, 0, :, :]], axis=1)
    ai = jnp.concatenate([ai0_ref[0, 0, :, :], ai1_ref[0, 0, :, :]], axis=1)
    ou_ref[:, :] = 0.25 * wu_ref[:, :] + dinvu * au
    oi_ref[:, :] = 0.25 * wi_ref[:, :] + dinvi * ai


_blend = pl.pallas_call(
    _blend_body,
    grid=(_NBLK,),
    in_specs=[
        pl.BlockSpec((_BLK, _D), lambda i: (i, 0)),
        pl.BlockSpec((_BLK, _D), lambda i: (i + _NBLK, 0)),
        pl.BlockSpec((1, _BLK, 16), lambda i: (0, i, 0)),
        pl.BlockSpec((1, _BLK, 16), lambda i: (1, i, 0)),
        pl.BlockSpec((1, 1, _BLK, _DH), lambda i: (0, 0, i, 0)),
        pl.BlockSpec((1, 1, _BLK, _DH), lambda i: (0, 1, i, 0)),
        pl.BlockSpec((1, 1, _BLK, _DH), lambda i: (1, 0, i, 0)),
        pl.BlockSpec((1, 1, _BLK, _DH), lambda i: (1, 1, i, 0)),
    ],
    out_specs=[
        pl.BlockSpec((_BLK, _D), lambda i: (i, 0)),
        pl.BlockSpec((_BLK, _D), lambda i: (i, 0)),
    ],
    out_shape=[
        jax.ShapeDtypeStruct((_NU, _D), jnp.float32),
        jax.ShapeDtypeStruct((_NI, _D), jnp.float32),
    ],
)


def kernel(embed_weight, user_idxs, user_id_idx, item_id_idx):
    del user_idxs
    w = embed_weight
    npad = _EPAD - _E
    pad_s = jnp.full((npad,), _TRASH, jnp.int32)
    sidx = jnp.stack([
        jnp.concatenate([user_id_idx, pad_s]),
        jnp.concatenate([item_id_idx, pad_s]),
    ]).reshape(2, _ROWS_PER_CORE, _CHUNK)

    dinv = _hist(sidx)
    acc = _spmm(sidx, w, dinv)
    user_embed, item_embed = _blend(w, w, dinv, dinv, acc, acc, acc, acc)
    return (user_embed, item_embed)
